# baseline (speedup 1.0000x reference)
import functools

import jax
import jax.numpy as jnp
from jax import lax
from jax.experimental import pallas as pl
from jax.experimental.pallas import tpu as pltpu
import jax.experimental.pallas.tpu_sc as plsc

_NC, _NS = 2, 16
_NW = _NC * _NS
_SEG = 20
_R2 = 2560
_KP = 24
_WS = 4
_MTAB_ROWS = 7168


def _seg_matrix(n_seg, rows_per_seg, n_rows):
    r = lax.broadcasted_iota(jnp.int32, (n_seg, n_rows), 1)
    s = lax.broadcasted_iota(jnp.int32, (n_seg, n_rows), 0)
    lo = s * rows_per_seg
    return ((r >= lo) & (r < lo + rows_per_seg)).astype(jnp.float32)


def _ln(x, g, b):
    mu = jnp.mean(x, axis=-1, keepdims=True)
    var = jnp.mean((x - mu) ** 2, axis=-1, keepdims=True)
    return (x - mu) * lax.rsqrt(var + 1e-5) * g + b


def _lnT(x, g, b):
    mu = jnp.mean(x, axis=0, keepdims=True)
    var = jnp.mean((x - mu) ** 2, axis=0, keepdims=True)
    return (x - mu) * lax.rsqrt(var + 1e-5) * g + b


def _dotT(x, w):
    return lax.dot_general(x, w, (((0,), (0,)), ((), ())),
                           preferred_element_type=jnp.float32)


def _gseg(n_seg, per, n, base):
    g = lax.broadcasted_iota(jnp.int32, (n_seg, n), 1) + base
    lo = lax.broadcasted_iota(jnp.int32, (n_seg, n), 0) * per
    return ((g >= lo) & (g < lo + per)).astype(jnp.float32)


def _k1_body(t_per, m_per, n_m, n_t, tl, ml, B,
             xt_ref, xm_ref, ln_t_g, ln_t_b, W_t1, b_t1,
             ln_m_g, ln_m_b, W_m1, b_m1, W_m2, b_m2,
             m_out, St_ref, Sm_ref):
    i = pl.program_id(0)
    xn = _lnT(xt_ref[...], ln_t_g[...], ln_t_b[...])
    h1t = jnp.maximum(_dotT(xn, W_t1[...]) + b_t1[...], 0.0)
    gidx = lax.broadcasted_iota(jnp.int32, (tl, 1), 0) + i * tl
    h1t = jnp.where(gidx < n_t, h1t, 0.0)
    part_t = jax.lax.dot(_gseg(B, t_per, tl, i * tl), h1t,
                         precision=lax.Precision.HIGHEST,
                         preferred_element_type=jnp.float32)

    xmn = _lnT(xm_ref[...], ln_m_g[...], ln_m_b[...])
    h1m = jnp.maximum(_dotT(xmn, W_m1[...]) + b_m1[...], 0.0)
    part_m = jax.lax.dot(_gseg(B, m_per, ml, i * ml), h1m,
                         precision=lax.Precision.HIGHEST,
                         preferred_element_type=jnp.float32)

    @pl.when(i == 0)
    def _():
        St_ref[...] = part_t
        Sm_ref[...] = part_m

    @pl.when(i > 0)
    def _():
        St_ref[...] += part_t
        Sm_ref[...] += part_m

    rows = lax.broadcasted_iota(jnp.int32, (ml, 1), 0) + i * ml
    m_out[:, 0:64] = jnp.where(rows < n_m,
                               h1m @ W_m2[...] + b_m2[...], 0.0)


def _stage1(xT, xmT, ln_t_g, ln_t_b, W_t1, b_t1,
            ln_m_g, ln_m_b, W_m1, b_m1, W_m2, b_m2, B, t_per, m_per, H,
            n_m):
    D_T, N_T = xT.shape
    D_M = xmT.shape[0]
    g1 = 8
    tl = 12800
    ml = _MTAB_ROWS // g1
    full = lambda a: pl.BlockSpec(a.shape, lambda i: (0,) * a.ndim)
    return pl.pallas_call(
        functools.partial(_k1_body, t_per, m_per, n_m, N_T, tl, ml, B),
        grid=(g1,),
        in_specs=[
            pl.BlockSpec((D_T, tl), lambda i: (0, i)),
            pl.BlockSpec((D_M, ml), lambda i: (0, i)),
            full(ln_t_g), full(ln_t_b), full(W_t1), full(b_t1),
            full(ln_m_g), full(ln_m_b), full(W_m1), full(b_m1),
            full(W_m2), full(b_m2),
        ],
        out_specs=[
            pl.BlockSpec((ml, 128), lambda i: (i, 0)),
            pl.BlockSpec((B, H), lambda i: (0, 0)),
            pl.BlockSpec((B, H), lambda i: (0, 0)),
        ],
        out_shape=[
            jax.ShapeDtypeStruct((_MTAB_ROWS, 128), jnp.float32),
            jax.ShapeDtypeStruct((B, H), jnp.float32),
            jax.ShapeDtypeStruct((B, H), jnp.float32),
        ],
    )(xT, xmT, ln_t_g, ln_t_b, W_t1, b_t1,
      ln_m_g, ln_m_b, W_m1, b_m1, W_m2, b_m2)


@functools.lru_cache(maxsize=None)
def _make_sc_a(n_pad, n_flat, d_t, m_per, null_row):
    rows_w = n_pad // _NW
    flat_w = rows_w * d_t
    mesh = plsc.VectorSubcoreMesh(core_axis_name="c", subcore_axis_name="s",
                                  num_cores=_NC, num_subcores=_NS)

    @functools.partial(
        pl.kernel,
        out_type=(jax.ShapeDtypeStruct((n_pad * _KP,), jnp.float32),
                  jax.ShapeDtypeStruct((n_pad,), jnp.int32)),
        mesh=mesh,
        scratch_types=[
            pltpu.VMEM((rows_w,), jnp.int32),
            pltpu.VMEM((rows_w,), jnp.int32),
            pltpu.VMEM((flat_w,), jnp.int32),
            pltpu.VMEM((flat_w,), jnp.float32),
            pltpu.VMEM((rows_w,), jnp.int32),
            pltpu.SemaphoreType.DMA,
        ],
        compiler_params=pltpu.CompilerParams(use_tc_tiling_on_sc=False),
    )
    def sc(xflat_hbm, b_hbm, idx_hbm, xg_out, tm_out,
           idx_v, b_v, iall_v, xcols_v, tm_v, sem):
        wid = lax.axis_index("s") * _NC + lax.axis_index("c")
        base = wid * rows_w
        pltpu.sync_copy(idx_hbm.at[pl.ds(base, rows_w)], idx_v)
        pltpu.sync_copy(b_hbm.at[pl.ds(base, rows_w)], b_v)
        n_rows = n_flat // d_t
        for c in range(rows_w // 16):
            t = idx_v[pl.ds(c * 16, 16)]
            for k in range(d_t):
                iall_v[pl.ds(k * rows_w + c * 16, 16)] = t + k * n_rows
        cps = [pltpu.async_copy(
                   xflat_hbm.at[iall_v.at[pl.ds(q * 128, 128)]],
                   xcols_v.at[pl.ds(q * 128, 128)], sem)
               for q in range(flat_w // 128)]
        for cp in cps:
            cp.wait()
        for c in range(rows_w // 16):
            col0 = xcols_v[pl.ds(c * 16, 16)]
            bb = b_v[pl.ds(c * 16, 16)]
            idxc = idx_v[pl.ds(c * 16, 16)]
            tm = jnp.where(col0 == -1.0,
                           null_row + (idxc & 7),
                           col0.astype(jnp.int32) + m_per * bb)
            tm_v[pl.ds(c * 16, 16)] = tm
        pltpu.sync_copy(xcols_v, xg_out.at[pl.ds(base * _KP, flat_w)])
        pltpu.sync_copy(tm_v, tm_out.at[pl.ds(base, rows_w)])

    return sc


@functools.lru_cache(maxsize=None)
def _make_sc_b(n_pad, tab_rows):
    rows_w = n_pad // _NW
    mesh = plsc.VectorSubcoreMesh(core_axis_name="c", subcore_axis_name="s",
                                  num_cores=_NC, num_subcores=_NS)

    @functools.partial(
        pl.kernel,
        out_type=jax.ShapeDtypeStruct((n_pad, 128), jnp.float32),
        mesh=mesh,
        scratch_types=[
            pltpu.VMEM((rows_w,), jnp.int32),
            pltpu.VMEM((rows_w, 128), jnp.float32),
            pltpu.SemaphoreType.DMA,
        ],
    )
    def sc(tm_hbm, mtab_hbm, mg_out, tm_v, rows_v, sem):
        wid = lax.axis_index("s") * _NC + lax.axis_index("c")
        base = wid * rows_w
        pltpu.sync_copy(tm_hbm.at[pl.ds(base, rows_w)], tm_v)
        cps = [pltpu.async_copy(
                   mtab_hbm.at[tm_v.at[pl.ds(c * 128, 128)]],
                   rows_v.at[pl.ds(c * 128, 128)], sem)
               for c in range(rows_w // 128)]
        for cp in cps:
            cp.wait()
        pltpu.sync_copy(rows_v, mg_out.at[pl.ds(base, rows_w)])

    return sc


def _k2_body(t_per, m_per, B, H, n_m,
             xg_ref, mg_ref, b_ref, St_ref, Sm_ref,
             ln_t_g, ln_t_b, W_t1, b_t1, W_t2, b_t2, W_m2, b_m2,
             W_a1, b_a1, W_a2, b_a2, W_g1, b_g1, W_g2, b_g2,
             W_o1, b_o1, W_o2, b_o2, out_ref, aggr_s):
    i = pl.program_id(0)

    @pl.when(i == 0)
    def _():
        t_mean = (St_ref[...] * (1.0 / t_per)) @ W_t2[...] + b_t2[...]
        m_mean = (Sm_ref[0:B, :] * (1.0 / m_per)) @ W_m2[...] + b_m2[...]
        a = jnp.maximum(t_mean @ W_a1[:H, :] + m_mean @ W_a1[H:, :]
                        + b_a1[...], 0.0)
        aggr_s[...] = a @ W_a2[...] + b_a2[...]

    d_t = ln_t_g.shape[0]
    rows_w = _R2 // _WS
    parts = []
    for w in range(_WS):
        xk = xg_ref[w * _KP:w * _KP + d_t, :]
        mu = jnp.mean(xk, axis=0, keepdims=True)
        var = jnp.mean((xk - mu) ** 2, axis=0, keepdims=True)
        xn = ((xk - mu) * lax.rsqrt(var + 1e-5) * ln_t_g[...]
              + ln_t_b[...])
        h1w = jnp.maximum(
            lax.dot_general(xn, W_t1[...], (((0,), (0,)), ((), ())),
                            preferred_element_type=jnp.float32)
            + b_t1[...], 0.0)
        parts.append(h1w)
    h1 = jnp.concatenate(parts, axis=0)
    t_h = h1 @ W_t2[...] + b_t2[...]
    onehotT = (b_ref[0] ==
               lax.broadcasted_iota(jnp.int32, (B, _R2), 0)
               ).astype(jnp.float32)
    agT = lax.dot_general(aggr_s[...], onehotT, (((0,), (0,)), ((), ())),
                          preferred_element_type=jnp.float32)
    ag_c = lax.dot_general(agT, W_g1[H:2 * H, :], (((0,), (0,)), ((), ())),
                           preferred_element_type=jnp.float32)
    g = jnp.maximum(t_h @ W_g1[:H, :] + ag_c
                    + mg_ref[:, 0:64] @ W_g1[2 * H:, :] + b_g1[...], 0.0)
    h2 = g @ W_g2[...] + b_g2[...]
    o = jnp.maximum(h2 @ W_o1[...] + b_o1[...], 0.0) @ W_o2[...] + b_o2[...]
    out_ref[...] = o


def _stage3(xgT, mg, b_pad, S_t, S_m,
            ln_t_g, ln_t_b, W_t1, b_t1, W_t2, b_t2, W_m2, b_m2,
            W_a1, b_a1, W_a2, b_a2, W_g1, b_g1, W_g2, b_g2,
            W_o1, b_o1, W_o2, b_o2, t_per, m_per, B, H, n_m, n_lab):
    n_pad = b_pad.shape[0] * b_pad.shape[2]
    g3 = n_pad // _R2
    full = lambda a: pl.BlockSpec(a.shape, lambda i: (0,) * a.ndim)
    return pl.pallas_call(
        functools.partial(_k2_body, t_per, m_per, B, H, n_m),
        grid=(g3,),
        in_specs=[
            pl.BlockSpec((_WS * _KP, n_pad // _NW), lambda i: (i, 0)),
            pl.BlockSpec((_R2, 128), lambda i: (i, 0)),
            pl.BlockSpec((1, 1, _R2), lambda i: (i, 0, 0)),
            full(S_t), full(S_m),
            full(ln_t_g), full(ln_t_b), full(W_t1), full(b_t1),
            full(W_t2), full(b_t2), full(W_m2), full(b_m2),
            full(W_a1), full(b_a1), full(W_a2), full(b_a2),
            full(W_g1), full(b_g1), full(W_g2), full(b_g2),
            full(W_o1), full(b_o1), full(W_o2), full(b_o2),
        ],
        out_specs=pl.BlockSpec((_R2, 1), lambda i: (i, 0)),
        out_shape=jax.ShapeDtypeStruct((n_lab, 1), jnp.float32),
        scratch_shapes=[pltpu.VMEM((B, H), jnp.float32)],
    )(xgT, mg, b_pad, S_t, S_m,
      ln_t_g, ln_t_b, W_t1, b_t1, W_t2, b_t2, W_m2, b_m2,
      W_a1, b_a1, W_a2, b_a2, W_g1, b_g1, W_g2, b_g2,
      W_o1, b_o1, W_o2, b_o2)


def _sc_a(xflatT, d_t, b_pad, idx_pad, m_per, null_row):
    n_pad = idx_pad.shape[0]
    fn = _make_sc_a(n_pad, xflatT.shape[0], d_t, m_per, null_row)
    return fn(xflatT, b_pad, idx_pad)


def _sc_b(tm, m_tab):
    fn = _make_sc_b(tm.shape[0], m_tab.shape[0])
    return fn(tm, m_tab)


def kernel(x_tasks, x_machines, x_tasks_batch, x_machines_batch,
           task_label_idx,
           ln_t_g, ln_t_b, W_t1, b_t1, W_t2, b_t2,
           ln_m_g, ln_m_b, W_m1, b_m1, W_m2, b_m2,
           W_a1, b_a1, W_a2, b_a2, W_g1, b_g1, W_g2, b_g2,
           W_o1, b_o1, W_o2, b_o2):
    N_T, D_T = x_tasks.shape
    N_M = x_machines.shape[0]
    B = x_machines_batch.shape[0] // 50
    t_per = N_T // B
    m_per = N_M // B
    H = W_t1.shape[1]

    n_lab = task_label_idx.shape[0]
    chunk = _NW * 128
    n_pad = ((n_lab + chunk - 1) // chunk) * chunk
    pad_idx = (jnp.arange(n_pad - n_lab, dtype=jnp.int32) * 41) % N_T
    idx_pad = jnp.concatenate([task_label_idx, pad_idx])
    b_pad = idx_pad // t_per

    xT = x_tasks.T
    xmT = jnp.concatenate(
        [x_machines.T,
         jnp.zeros((x_machines.shape[1], _MTAB_ROWS - N_M), jnp.float32)],
        axis=1)
    xflatT = xT.reshape(-1)

    xg_flat, tm = _sc_a(xflatT, D_T, b_pad, idx_pad, m_per, N_M)
    xgT = xg_flat.reshape(_NW * _KP, n_pad // _NW)

    m_tab, S_t, S_m = _stage1(
        xT, xmT, ln_t_g.reshape(D_T, 1), ln_t_b.reshape(D_T, 1),
        W_t1, b_t1,
        ln_m_g.reshape(-1, 1), ln_m_b.reshape(-1, 1),
        W_m1, b_m1, W_m2, b_m2, B, t_per, m_per, H, N_M)

    mg = _sc_b(tm, m_tab)

    return _stage3(xgT, mg, b_pad.reshape(n_pad // _R2, 1, _R2), S_t, S_m,
                   ln_t_g.reshape(D_T, 1), ln_t_b.reshape(D_T, 1),
                   W_t1, b_t1, W_t2, b_t2, W_m2, b_m2,
                   W_a1, b_a1, W_a2, b_a2, W_g1, b_g1, W_g2, b_g2,
                   W_o1, b_o1, W_o2, b_o2, t_per, m_per, B, H, N_M, n_lab)


# speedup vs baseline: 1.4318x; 1.4318x over previous
import functools

import jax
import jax.numpy as jnp
from jax import lax
from jax.experimental import pallas as pl
from jax.experimental.pallas import tpu as pltpu
import jax.experimental.pallas.tpu_sc as plsc

_NC, _NS = 2, 16
_NW = _NC * _NS
_SEG = 20
_R2 = 2560
_KP = 24
_WS = 4
_MTAB_ROWS = 7168


def _seg_matrix(n_seg, rows_per_seg, n_rows):
    r = lax.broadcasted_iota(jnp.int32, (n_seg, n_rows), 1)
    s = lax.broadcasted_iota(jnp.int32, (n_seg, n_rows), 0)
    lo = s * rows_per_seg
    return ((r >= lo) & (r < lo + rows_per_seg)).astype(jnp.float32)


def _ln(x, g, b):
    mu = jnp.mean(x, axis=-1, keepdims=True)
    var = jnp.mean((x - mu) ** 2, axis=-1, keepdims=True)
    return (x - mu) * lax.rsqrt(var + 1e-5) * g + b


def _lnT(x, g, b):
    mu = jnp.mean(x, axis=0, keepdims=True)
    var = jnp.mean((x - mu) ** 2, axis=0, keepdims=True)
    return (x - mu) * lax.rsqrt(var + 1e-5) * g + b


def _dotT(x, w):
    return lax.dot_general(x, w, (((0,), (0,)), ((), ())),
                           preferred_element_type=jnp.float32)


def _gseg(n_seg, per, n, base):
    g = lax.broadcasted_iota(jnp.int32, (n_seg, n), 1) + base
    lo = lax.broadcasted_iota(jnp.int32, (n_seg, n), 0) * per
    return ((g >= lo) & (g < lo + per)).astype(jnp.float32)


def _k1_body(t_per, m_per, n_m, n_t, tl, ml, B,
             xt_ref, xm_ref, ln_t_g, ln_t_b, W_t1, b_t1,
             ln_m_g, ln_m_b, W_m1, b_m1, W_m2, b_m2,
             m_out, St_ref, Sm_ref):
    i = pl.program_id(0)
    xn = _lnT(xt_ref[...], ln_t_g[...], ln_t_b[...])
    h1t = jnp.maximum(_dotT(xn, W_t1[...]) + b_t1[...], 0.0)
    gidx = lax.broadcasted_iota(jnp.int32, (tl, 1), 0) + i * tl
    h1t = jnp.where(gidx < n_t, h1t, 0.0)
    part_t = jax.lax.dot(_gseg(B, t_per, tl, i * tl), h1t,
                         preferred_element_type=jnp.float32)

    xmn = _lnT(xm_ref[...], ln_m_g[...], ln_m_b[...])
    h1m = jnp.maximum(_dotT(xmn, W_m1[...]) + b_m1[...], 0.0)
    part_m = jax.lax.dot(_gseg(B, m_per, ml, i * ml), h1m,
                         preferred_element_type=jnp.float32)

    @pl.when(i == 0)
    def _():
        St_ref[...] = part_t
        Sm_ref[...] = part_m

    @pl.when(i > 0)
    def _():
        St_ref[...] += part_t
        Sm_ref[...] += part_m

    rows = lax.broadcasted_iota(jnp.int32, (ml, 1), 0) + i * ml
    m_out[:, 0:64] = jnp.where(rows < n_m,
                               h1m @ W_m2[...] + b_m2[...], 0.0)


def _stage1(xT, xmT, ln_t_g, ln_t_b, W_t1, b_t1,
            ln_m_g, ln_m_b, W_m1, b_m1, W_m2, b_m2, B, t_per, m_per, H,
            n_m):
    D_T, N_T = xT.shape
    D_M = xmT.shape[0]
    g1 = 8
    tl = 12800
    ml = _MTAB_ROWS // g1
    full = lambda a: pl.BlockSpec(a.shape, lambda i: (0,) * a.ndim)
    return pl.pallas_call(
        functools.partial(_k1_body, t_per, m_per, n_m, N_T, tl, ml, B),
        grid=(g1,),
        in_specs=[
            pl.BlockSpec((D_T, tl), lambda i: (0, i)),
            pl.BlockSpec((D_M, ml), lambda i: (0, i)),
            full(ln_t_g), full(ln_t_b), full(W_t1), full(b_t1),
            full(ln_m_g), full(ln_m_b), full(W_m1), full(b_m1),
            full(W_m2), full(b_m2),
        ],
        out_specs=[
            pl.BlockSpec((ml, 128), lambda i: (i, 0)),
            pl.BlockSpec((B, H), lambda i: (0, 0)),
            pl.BlockSpec((B, H), lambda i: (0, 0)),
        ],
        out_shape=[
            jax.ShapeDtypeStruct((_MTAB_ROWS, 128), jnp.float32),
            jax.ShapeDtypeStruct((B, H), jnp.float32),
            jax.ShapeDtypeStruct((B, H), jnp.float32),
        ],
    )(xT, xmT, ln_t_g, ln_t_b, W_t1, b_t1,
      ln_m_g, ln_m_b, W_m1, b_m1, W_m2, b_m2)


@functools.lru_cache(maxsize=None)
def _make_sc_a(n_pad, n_flat, d_t, m_per, null_row):
    rows_w = n_pad // _NW
    flat_w = rows_w * d_t
    mesh = plsc.VectorSubcoreMesh(core_axis_name="c", subcore_axis_name="s",
                                  num_cores=_NC, num_subcores=_NS)

    @functools.partial(
        pl.kernel,
        out_type=(jax.ShapeDtypeStruct((n_pad * _KP,), jnp.float32),
                  jax.ShapeDtypeStruct((n_pad,), jnp.int32)),
        mesh=mesh,
        scratch_types=[
            pltpu.VMEM((rows_w,), jnp.int32),
            pltpu.VMEM((rows_w,), jnp.int32),
            pltpu.VMEM((flat_w,), jnp.int32),
            pltpu.VMEM((flat_w,), jnp.float32),
            pltpu.VMEM((rows_w,), jnp.int32),
            pltpu.SemaphoreType.DMA,
        ],
        compiler_params=pltpu.CompilerParams(use_tc_tiling_on_sc=False),
    )
    def sc(xflat_hbm, b_hbm, idx_hbm, xg_out, tm_out,
           idx_v, b_v, iall_v, xcols_v, tm_v, sem):
        wid = lax.axis_index("s") * _NC + lax.axis_index("c")
        base = wid * rows_w
        pltpu.sync_copy(idx_hbm.at[pl.ds(base, rows_w)], idx_v)
        pltpu.sync_copy(b_hbm.at[pl.ds(base, rows_w)], b_v)
        n_rows = n_flat // d_t
        for c in range(rows_w // 16):
            t = idx_v[pl.ds(c * 16, 16)]
            for k in range(d_t):
                iall_v[pl.ds(k * rows_w + c * 16, 16)] = t + k * n_rows
        cps = [pltpu.async_copy(
                   xflat_hbm.at[iall_v.at[pl.ds(q * 128, 128)]],
                   xcols_v.at[pl.ds(q * 128, 128)], sem)
               for q in range(flat_w // 128)]
        for cp in cps:
            cp.wait()
        for c in range(rows_w // 16):
            col0 = xcols_v[pl.ds(c * 16, 16)]
            bb = b_v[pl.ds(c * 16, 16)]
            idxc = idx_v[pl.ds(c * 16, 16)]
            tm = jnp.where(col0 == -1.0,
                           null_row + (idxc & 7),
                           col0.astype(jnp.int32) + m_per * bb)
            tm_v[pl.ds(c * 16, 16)] = tm
        pltpu.sync_copy(xcols_v, xg_out.at[pl.ds(base * _KP, flat_w)])
        pltpu.sync_copy(tm_v, tm_out.at[pl.ds(base, rows_w)])

    return sc


@functools.lru_cache(maxsize=None)
def _make_sc_b(n_pad, tab_rows):
    rows_w = n_pad // _NW
    mesh = plsc.VectorSubcoreMesh(core_axis_name="c", subcore_axis_name="s",
                                  num_cores=_NC, num_subcores=_NS)

    @functools.partial(
        pl.kernel,
        out_type=jax.ShapeDtypeStruct((n_pad, 128), jnp.float32),
        mesh=mesh,
        scratch_types=[
            pltpu.VMEM((rows_w,), jnp.int32),
            pltpu.VMEM((rows_w, 128), jnp.float32),
            pltpu.SemaphoreType.DMA,
        ],
    )
    def sc(tm_hbm, mtab_hbm, mg_out, tm_v, rows_v, sem):
        wid = lax.axis_index("s") * _NC + lax.axis_index("c")
        base = wid * rows_w
        pltpu.sync_copy(tm_hbm.at[pl.ds(base, rows_w)], tm_v)
        cps = [pltpu.async_copy(
                   mtab_hbm.at[tm_v.at[pl.ds(c * 128, 128)]],
                   rows_v.at[pl.ds(c * 128, 128)], sem)
               for c in range(rows_w // 128)]
        for cp in cps:
            cp.wait()
        pltpu.sync_copy(rows_v, mg_out.at[pl.ds(base, rows_w)])

    return sc


def _k2_body(t_per, m_per, B, H, n_m,
             xg_ref, mg_ref, b_ref, St_ref, Sm_ref,
             ln_t_g, ln_t_b, W_t1, b_t1, W_t2, b_t2, W_m2, b_m2,
             W_a1, b_a1, W_a2, b_a2, W_g1, b_g1, W_g2, b_g2,
             W_o1, b_o1, W_o2, b_o2, out_ref, aggr_s):
    i = pl.program_id(0)

    @pl.when(i == 0)
    def _():
        t_mean = (St_ref[...] * (1.0 / t_per)) @ W_t2[...] + b_t2[...]
        m_mean = (Sm_ref[0:B, :] * (1.0 / m_per)) @ W_m2[...] + b_m2[...]
        a = jnp.maximum(t_mean @ W_a1[:H, :] + m_mean @ W_a1[H:, :]
                        + b_a1[...], 0.0)
        aggr_s[...] = a @ W_a2[...] + b_a2[...]

    d_t = ln_t_g.shape[0]
    rows_w = _R2 // _WS
    parts = []
    for w in range(_WS):
        xk = xg_ref[w * _KP:w * _KP + d_t, :]
        mu = jnp.mean(xk, axis=0, keepdims=True)
        var = jnp.mean((xk - mu) ** 2, axis=0, keepdims=True)
        xn = ((xk - mu) * lax.rsqrt(var + 1e-5) * ln_t_g[...]
              + ln_t_b[...])
        h1w = jnp.maximum(
            lax.dot_general(xn, W_t1[...], (((0,), (0,)), ((), ())),
                            preferred_element_type=jnp.float32)
            + b_t1[...], 0.0)
        parts.append(h1w)
    h1 = jnp.concatenate(parts, axis=0)
    t_h = h1 @ W_t2[...] + b_t2[...]
    onehotT = (b_ref[0] ==
               lax.broadcasted_iota(jnp.int32, (B, _R2), 0)
               ).astype(jnp.float32)
    agT = lax.dot_general(aggr_s[...], onehotT, (((0,), (0,)), ((), ())),
                          preferred_element_type=jnp.float32)
    ag_c = lax.dot_general(agT, W_g1[H:2 * H, :], (((0,), (0,)), ((), ())),
                           preferred_element_type=jnp.float32)
    g = jnp.maximum(t_h @ W_g1[:H, :] + ag_c
                    + mg_ref[:, 0:64] @ W_g1[2 * H:, :] + b_g1[...], 0.0)
    h2 = g @ W_g2[...] + b_g2[...]
    o = jnp.maximum(h2 @ W_o1[...] + b_o1[...], 0.0) @ W_o2[...] + b_o2[...]
    out_ref[...] = o


def _stage3(xgT, mg, b_pad, S_t, S_m,
            ln_t_g, ln_t_b, W_t1, b_t1, W_t2, b_t2, W_m2, b_m2,
            W_a1, b_a1, W_a2, b_a2, W_g1, b_g1, W_g2, b_g2,
            W_o1, b_o1, W_o2, b_o2, t_per, m_per, B, H, n_m, n_lab):
    n_pad = b_pad.shape[0] * b_pad.shape[2]
    g3 = n_pad // _R2
    full = lambda a: pl.BlockSpec(a.shape, lambda i: (0,) * a.ndim)
    return pl.pallas_call(
        functools.partial(_k2_body, t_per, m_per, B, H, n_m),
        grid=(g3,),
        in_specs=[
            pl.BlockSpec((_WS * _KP, n_pad // _NW), lambda i: (i, 0)),
            pl.BlockSpec((_R2, 128), lambda i: (i, 0)),
            pl.BlockSpec((1, 1, _R2), lambda i: (i, 0, 0)),
            full(S_t), full(S_m),
            full(ln_t_g), full(ln_t_b), full(W_t1), full(b_t1),
            full(W_t2), full(b_t2), full(W_m2), full(b_m2),
            full(W_a1), full(b_a1), full(W_a2), full(b_a2),
            full(W_g1), full(b_g1), full(W_g2), full(b_g2),
            full(W_o1), full(b_o1), full(W_o2), full(b_o2),
        ],
        out_specs=pl.BlockSpec((_R2, 1), lambda i: (i, 0)),
        out_shape=jax.ShapeDtypeStruct((n_lab, 1), jnp.float32),
        scratch_shapes=[pltpu.VMEM((B, H), jnp.float32)],
    )(xgT, mg, b_pad, S_t, S_m,
      ln_t_g, ln_t_b, W_t1, b_t1, W_t2, b_t2, W_m2, b_m2,
      W_a1, b_a1, W_a2, b_a2, W_g1, b_g1, W_g2, b_g2,
      W_o1, b_o1, W_o2, b_o2)


def _sc_a(xflatT, d_t, b_pad, idx_pad, m_per, null_row):
    n_pad = idx_pad.shape[0]
    fn = _make_sc_a(n_pad, xflatT.shape[0], d_t, m_per, null_row)
    return fn(xflatT, b_pad, idx_pad)


def _sc_b(tm, m_tab):
    fn = _make_sc_b(tm.shape[0], m_tab.shape[0])
    return fn(tm, m_tab)


def kernel(x_tasks, x_machines, x_tasks_batch, x_machines_batch,
           task_label_idx,
           ln_t_g, ln_t_b, W_t1, b_t1, W_t2, b_t2,
           ln_m_g, ln_m_b, W_m1, b_m1, W_m2, b_m2,
           W_a1, b_a1, W_a2, b_a2, W_g1, b_g1, W_g2, b_g2,
           W_o1, b_o1, W_o2, b_o2):
    N_T, D_T = x_tasks.shape
    N_M = x_machines.shape[0]
    B = x_machines_batch.shape[0] // 50
    t_per = N_T // B
    m_per = N_M // B
    H = W_t1.shape[1]

    n_lab = task_label_idx.shape[0]
    chunk = _NW * 128
    n_pad = ((n_lab + chunk - 1) // chunk) * chunk
    pad_idx = (jnp.arange(n_pad - n_lab, dtype=jnp.int32) * 41) % N_T
    idx_pad = jnp.concatenate([task_label_idx, pad_idx])
    b_pad = idx_pad // t_per

    xT = x_tasks.T
    xmT = jnp.concatenate(
        [x_machines.T,
         jnp.zeros((x_machines.shape[1], _MTAB_ROWS - N_M), jnp.float32)],
        axis=1)
    xflatT = xT.reshape(-1)

    xg_flat, tm = _sc_a(xflatT, D_T, b_pad, idx_pad, m_per, N_M)
    xgT = xg_flat.reshape(_NW * _KP, n_pad // _NW)

    m_tab, S_t, S_m = _stage1(
        xT, xmT, ln_t_g.reshape(D_T, 1), ln_t_b.reshape(D_T, 1),
        W_t1, b_t1,
        ln_m_g.reshape(-1, 1), ln_m_b.reshape(-1, 1),
        W_m1, b_m1, W_m2, b_m2, B, t_per, m_per, H, N_M)

    mg = _sc_b(tm, m_tab)

    return _stage3(xgT, mg, b_pad.reshape(n_pad // _R2, 1, _R2), S_t, S_m,
                   ln_t_g.reshape(D_T, 1), ln_t_b.reshape(D_T, 1),
                   W_t1, b_t1, W_t2, b_t2, W_m2, b_m2,
                   W_a1, b_a1, W_a2, b_a2, W_g1, b_g1, W_g2, b_g2,
                   W_o1, b_o1, W_o2, b_o2, t_per, m_per, B, H, N_M, n_lab)
